# baseline (speedup 1.0000x reference)
import functools

import jax
import jax.numpy as jnp
from jax import lax
from jax.experimental import pallas as pl
from jax.experimental.pallas import tpu as pltpu
from jax.experimental.pallas import tpu_sc as plsc

_N = 10000
_E = 320000
_D = 128
_H1 = 128
_H2 = 128
_R = 8
_NB = 30

_NC = 2
_NS = 16
_NW = _NC * _NS
_CH = 128
_NCHUNK = _E // _CH
_Q, _REM = divmod(_NCHUNK, _NW)
_ROWS_PER_TILE = _N // _NS
_CW = 16
_NBUF = 3

_mesh = plsc.VectorSubcoreMesh(
    core_axis_name="c", subcore_axis_name="s", num_cores=_NC, num_subcores=_NS)
_sc_params = pltpu.CompilerParams(use_tc_tiling_on_sc=False,
                                  needs_layout_passes=False)


def _worker_range(c, s):
    w = s * _NC + c
    start = w * _Q + jnp.minimum(w, _REM)
    mycnt = _Q + (w < _REM).astype(jnp.int32)
    return start, mycnt


def _scz_body(cmb, zcnt, outc, acc_cnt, dstr0, dstr1, obuf,
              semr0, semr1, semo0, semo1, sem_ld):
    c = lax.axis_index("c")
    s = lax.axis_index("s")
    start, mycnt = _worker_range(c, s)
    rbase = s * _ROWS_PER_TILE

    pltpu.make_async_copy(zcnt.at[pl.ds(rbase, _ROWS_PER_TILE)],
                          acc_cnt.at[pl.ds(rbase, _ROWS_PER_TILE)],
                          sem_ld).start()

    def ones_row(i, carry):
        obuf[i, :] = jnp.ones((_CW,), jnp.float32)
        return carry
    lax.fori_loop(0, _CH, ones_row, 0)

    dstr = (dstr0, dstr1)
    rsems = (semr0, semr1)
    osems = (semo0, semo1)

    def rows(p, b):
        return pltpu.make_async_copy(cmb.at[p, 2], dstr[b], rsems[b])

    def scat_ones(b):
        return pltpu.make_async_copy(obuf, acc_cnt.at[dstr[b]], osems[b])

    pltpu.make_async_copy(zcnt.at[pl.ds(rbase, _ROWS_PER_TILE)],
                          acc_cnt.at[pl.ds(rbase, _ROWS_PER_TILE)],
                          sem_ld).wait()
    plsc.subcore_barrier()

    @pl.when(mycnt > 0)
    def _():
        rows(start, 0).start()
        rows(start, 0).wait()

    def chunk_body(j, carry):
        def arm(b):
            ob = 1 - b

            @pl.when(j >= 1)
            def _():
                scat_ones(ob).wait()

            @pl.when(j + 1 < mycnt)
            def _():
                rows(start + j + 1, ob).start()

            scat_ones(b).start(add=True)

            @pl.when(j + 1 < mycnt)
            def _():
                rows(start + j + 1, ob).wait()

        @pl.when(j % 2 == 0)
        def _():
            arm(0)

        @pl.when(j % 2 == 1)
        def _():
            arm(1)

        return carry

    lax.fori_loop(0, mycnt, chunk_body, 0)

    @pl.when(mycnt > 0)
    def _():
        @pl.when((mycnt - 1) % 2 == 0)
        def _():
            scat_ones(0).wait()

        @pl.when((mycnt - 1) % 2 == 1)
        def _():
            scat_ones(1).wait()

    plsc.subcore_barrier()
    pltpu.sync_copy(acc_cnt.at[pl.ds(rbase, _ROWS_PER_TILE)],
                    outc.at[c, pl.ds(rbase, _ROWS_PER_TILE)])


_sc_cnt = functools.partial(
    pl.kernel,
    out_type=jax.ShapeDtypeStruct((_NC, _N, _CW), jnp.float32),
    mesh=_mesh,
    compiler_params=_sc_params,
    scratch_types=[
        pltpu.VMEM_SHARED((_N, _CW), jnp.float32),
        pltpu.VMEM((_CH,), jnp.int32),
        pltpu.VMEM((_CH,), jnp.int32),
        pltpu.VMEM((_CH, _CW), jnp.float32),
        pltpu.SemaphoreType.DMA,
        pltpu.SemaphoreType.DMA,
        pltpu.SemaphoreType.DMA,
        pltpu.SemaphoreType.DMA,
        pltpu.SemaphoreType.DMA,
    ],
)(_scz_body)


def _dense_a_body(att_ref, basis_ref, x_ref, root_ref, b1_ref, xw_ref, xr_ref):
    r = pl.program_id(0)

    def bstep(b, acc):
        return acc + att_ref[r, b] * basis_ref[b]

    wr = lax.fori_loop(0, _NB, bstep, jnp.zeros((_D, _H1), jnp.float32))
    xw_ref[0] = jnp.dot(x_ref[...], wr, preferred_element_type=jnp.float32)

    @pl.when(r == 0)
    def _():
        xr_ref[...] = (jnp.dot(x_ref[...], root_ref[...],
                               preferred_element_type=jnp.float32)
                       + b1_ref[...])


_dense_a = pl.pallas_call(
    _dense_a_body,
    grid=(_R,),
    in_specs=[
        pl.BlockSpec(memory_space=pltpu.SMEM),
        pl.BlockSpec((_NB, _D, _H1), lambda r: (0, 0, 0)),
        pl.BlockSpec((_N, _D), lambda r: (0, 0)),
        pl.BlockSpec((_D, _H1), lambda r: (0, 0)),
        pl.BlockSpec((1, _H1), lambda r: (0, 0)),
    ],
    out_specs=[
        pl.BlockSpec((1, _N, _H1), lambda r: (r, 0, 0)),
        pl.BlockSpec((_N, _H1), lambda r: (0, 0)),
    ],
    out_shape=[
        jax.ShapeDtypeStruct((_R, _N, _H1), jnp.float32),
        jax.ShapeDtypeStruct((_N, _H1), jnp.float32),
    ],
)


def _ring_pass(body_scale, use_gidx):
    def _body(table, cmb, zsum, out,
              acc, cmb0, cmb1, cmb2,
              semr0, semr1, semr2, semg0, semg1, semg2,
              sems0, sems1, sems2, sem_ld):
        def _scoped(gbuf0, gbuf1, gbuf2):
            _body_inner(table, cmb, zsum, out, acc, cmb0, cmb1, cmb2,
                        gbuf0, gbuf1, gbuf2,
                        semr0, semr1, semr2, semg0, semg1, semg2,
                        sems0, sems1, sems2, sem_ld)
        pl.run_scoped(_scoped,
                      pltpu.VMEM((_CH, _H1), jnp.float32),
                      pltpu.VMEM((_CH, _H1), jnp.float32),
                      pltpu.VMEM((_CH, _H1), jnp.float32))

    def _body_inner(table, cmb, zsum, out,
                    acc, cmb0, cmb1, cmb2, gbuf0, gbuf1, gbuf2,
                    semr0, semr1, semr2, semg0, semg1, semg2,
                    sems0, sems1, sems2, sem_ld):
        c = lax.axis_index("c")
        s = lax.axis_index("s")
        start, mycnt = _worker_range(c, s)
        rbase = s * _ROWS_PER_TILE

        pltpu.make_async_copy(zsum.at[pl.ds(rbase, _ROWS_PER_TILE)],
                              acc.at[pl.ds(rbase, _ROWS_PER_TILE)],
                              sem_ld).start()

        cmbb = (cmb0, cmb1, cmb2)
        bufs = (gbuf0, gbuf1, gbuf2)
        rsems = (semr0, semr1, semr2)
        gsems = (semg0, semg1, semg2)
        ssems = (sems0, sems1, sems2)

        def rows(p, b):
            return pltpu.make_async_copy(cmb.at[p], cmbb[b], rsems[b])

        def gather(b):
            return pltpu.make_async_copy(table.at[cmbb[b].at[0]], bufs[b],
                                         gsems[b])

        def scat(b):
            return pltpu.make_async_copy(bufs[b], acc.at[cmbb[b].at[2]],
                                         ssems[b])

        def make_gidx(b):
            if not use_gidx:
                return
            for g in range(_CH // 16):
                sl = pl.ds(16 * g, 16)
                cmbb[b][0, sl] = cmbb[b][1, sl] * _N + cmbb[b][0, sl]

        pltpu.make_async_copy(zsum.at[pl.ds(rbase, _ROWS_PER_TILE)],
                              acc.at[pl.ds(rbase, _ROWS_PER_TILE)],
                              sem_ld).wait()
        plsc.subcore_barrier()

        @pl.when(mycnt > 0)
        def _():
            rows(start, 0).start()

            @pl.when(mycnt > 1)
            def _():
                rows(start + 1, 1).start()

            rows(start, 0).wait()
            make_gidx(0)
            gather(0).start()

            @pl.when(mycnt > 1)
            def _():
                rows(start + 1, 1).wait()
                make_gidx(1)
                gather(1).start()

        def chunk_body(j, carry):
            def arm(b):
                nb = (b + 2) % _NBUF

                @pl.when(j >= 1)
                def _():
                    scat(nb).wait()

                @pl.when(j + 2 < mycnt)
                def _():
                    rows(start + j + 2, nb).start()

                gather(b).wait()

                @pl.when(j + 2 < mycnt)
                def _():
                    rows(start + j + 2, nb).wait()
                    make_gidx(nb)
                    gather(nb).start()

                body_scale(bufs, cmbb, b)

                scat(b).start(add=True)

            @pl.when(j % _NBUF == 0)
            def _():
                arm(0)

            @pl.when(j % _NBUF == 1)
            def _():
                arm(1)

            @pl.when(j % _NBUF == 2)
            def _():
                arm(2)

            return carry

        lax.fori_loop(0, mycnt, chunk_body, 0)

        @pl.when(mycnt > 0)
        def _():
            for b in range(_NBUF):
                @pl.when((mycnt - 1) % _NBUF == b)
                def _(b=b):
                    scat(b).wait()

        plsc.subcore_barrier()
        pltpu.sync_copy(acc.at[pl.ds(rbase, _ROWS_PER_TILE)],
                        out.at[c, pl.ds(rbase, _ROWS_PER_TILE)])

    return _body


def _scale_by_norm(bufs, cmbb, b):
    def egroup(q, c2):
        nv = plsc.bitcast(cmbb[b][3, pl.ds(16 * q, 16)], jnp.float32)
        for i in range(16):
            nrm = nv[i]
            e = q * 16 + i
            for g in range(_H1 // 16):
                sl = pl.ds(16 * g, 16)
                bufs[b][e, sl] = bufs[b][e, sl] * nrm
        return c2
    lax.fori_loop(0, _CH // 16, egroup, 0)


def _no_scale(bufs, cmbb, b):
    pass


_ring_scratch = [
    pltpu.VMEM_SHARED((_N, _H1), jnp.float32),
    pltpu.VMEM((4, _CH), jnp.int32),
    pltpu.VMEM((4, _CH), jnp.int32),
    pltpu.VMEM((4, _CH), jnp.int32),
    pltpu.SemaphoreType.DMA,
    pltpu.SemaphoreType.DMA,
    pltpu.SemaphoreType.DMA,
    pltpu.SemaphoreType.DMA,
    pltpu.SemaphoreType.DMA,
    pltpu.SemaphoreType.DMA,
    pltpu.SemaphoreType.DMA,
    pltpu.SemaphoreType.DMA,
    pltpu.SemaphoreType.DMA,
    pltpu.SemaphoreType.DMA,
]

_sc_pass1 = functools.partial(
    pl.kernel,
    out_type=jax.ShapeDtypeStruct((_NC, _N, _H1), jnp.float32),
    mesh=_mesh,
    compiler_params=_sc_params,
    scratch_types=list(_ring_scratch),
)(_ring_pass(_scale_by_norm, use_gidx=True))

_sc_pass2 = functools.partial(
    pl.kernel,
    out_type=jax.ShapeDtypeStruct((_NC, _N, _H2), jnp.float32),
    mesh=_mesh,
    compiler_params=_sc_params,
    scratch_types=list(_ring_scratch),
)(_ring_pass(_no_scale, use_gidx=False))


def _dense_c_body(p_ref, c_ref, xr_ref, wn_ref, wl_ref, b2_ref, h_ref, y2_ref):
    sums = p_ref[0] + p_ref[1]
    cnt = c_ref[0] + c_ref[1]
    cnt0 = jnp.max(cnt, axis=1, keepdims=True)
    x1 = sums / jnp.maximum(cnt0, 1.0) + xr_ref[...]
    h_ref[...] = jnp.dot(x1, wn_ref[...], preferred_element_type=jnp.float32)
    y2_ref[...] = (jnp.dot(x1, wl_ref[...], preferred_element_type=jnp.float32)
                   + b2_ref[...])


_dense_c = pl.pallas_call(
    _dense_c_body,
    out_shape=[
        jax.ShapeDtypeStruct((_N, _H1), jnp.float32),
        jax.ShapeDtypeStruct((_N, _H2), jnp.float32),
    ],
)


def _dense_e_body(q_ref, y2_ref, o_ref):
    o_ref[...] = q_ref[0] + q_ref[1] + y2_ref[...]


_dense_e = pl.pallas_call(
    _dense_e_body,
    out_shape=jax.ShapeDtypeStruct((_N, _H2), jnp.float32),
)


def kernel(node_features, edge_index, edge_type, edge_norm, basis, att, root,
           bias1, w_nbr, w_lin, bias2):
    src2 = edge_index[0].reshape(_NCHUNK, 1, _CH)
    typ2 = edge_type.reshape(_NCHUNK, 1, _CH)
    dst2 = edge_index[1].reshape(_NCHUNK, 1, _CH)
    norm_bits = lax.bitcast_convert_type(edge_norm, jnp.int32)
    norm2 = norm_bits.reshape(_NCHUNK, 1, _CH)
    cmb = jnp.concatenate([src2, typ2, dst2, norm2], axis=1)
    zsum = jnp.zeros((_N, _H1), jnp.float32)
    zcnt = jnp.zeros((_N, _CW), jnp.float32)

    cnt1 = _sc_cnt(cmb, zcnt)
    xw, xr = _dense_a(att, basis, node_features, root, bias1.reshape(1, _H1))
    xw_flat = xw.reshape(_R * _N, _H1)
    part1 = _sc_pass1(xw_flat, cmb, zsum)
    h, y2 = _dense_c(part1, cnt1, xr, w_nbr, w_lin, bias2.reshape(1, _H2))
    part2 = _sc_pass2(h, cmb, zsum)
    return _dense_e(part2, y2)


# speedup vs baseline: 1.0370x; 1.0370x over previous
import functools

import jax
import jax.numpy as jnp
from jax import lax
from jax.experimental import pallas as pl
from jax.experimental.pallas import tpu as pltpu
from jax.experimental.pallas import tpu_sc as plsc

_N = 10000
_E = 320000
_D = 128
_H1 = 128
_H2 = 128
_R = 8
_NB = 30

_NC = 2
_NS = 16
_NW = _NC * _NS
_CH = 128
_NCHUNK = _E // _CH
_Q, _REM = divmod(_NCHUNK, _NW)
_ROWS_PER_TILE = _N // _NS
_CW = 16
_NBUF = 3
_K = 1

_mesh = plsc.VectorSubcoreMesh(
    core_axis_name="c", subcore_axis_name="s", num_cores=_NC, num_subcores=_NS)
_sc_params = pltpu.CompilerParams(use_tc_tiling_on_sc=False,
                                  needs_layout_passes=False)


def _worker_range(c, s):
    w = s * _NC + c
    start = w * _Q + jnp.minimum(w, _REM)
    mycnt = _Q + (w < _REM).astype(jnp.int32)
    return start, mycnt


def _dense_a_body(att_ref, basis_ref, x_ref, root_ref, b1_ref, xw_ref, xr_ref):
    r = pl.program_id(0)

    def bstep(b, acc):
        return acc + att_ref[r, b] * basis_ref[b]

    wr = lax.fori_loop(0, _NB, bstep, jnp.zeros((_D, _H1), jnp.float32))
    xw_ref[0] = jnp.dot(x_ref[...], wr, preferred_element_type=jnp.float32)

    @pl.when(r == 0)
    def _():
        xr_ref[...] = (jnp.dot(x_ref[...], root_ref[...],
                               preferred_element_type=jnp.float32)
                       + b1_ref[...])


_dense_a = pl.pallas_call(
    _dense_a_body,
    grid=(_R,),
    in_specs=[
        pl.BlockSpec(memory_space=pltpu.SMEM),
        pl.BlockSpec((_NB, _D, _H1), lambda r: (0, 0, 0)),
        pl.BlockSpec((_N, _D), lambda r: (0, 0)),
        pl.BlockSpec((_D, _H1), lambda r: (0, 0)),
        pl.BlockSpec((1, _H1), lambda r: (0, 0)),
    ],
    out_specs=[
        pl.BlockSpec((1, _N, _H1), lambda r: (r, 0, 0)),
        pl.BlockSpec((_N, _H1), lambda r: (0, 0)),
    ],
    out_shape=[
        jax.ShapeDtypeStruct((_R, _N, _H1), jnp.float32),
        jax.ShapeDtypeStruct((_N, _H1), jnp.float32),
    ],
)


def _sc1_body(table, cmb, zsum, zcnt, outs, outc,
              acc, acc_cnt,
              cmb0, cmb1, gidx0, gidx1, gbuf0, gbuf1, obuf,
              semr0, semr1, semg0, semg1, sems0, sems1, semo, sem_ld):
    c = lax.axis_index("c")
    s = lax.axis_index("s")
    start, mycnt = _worker_range(c, s)
    rbase = s * _ROWS_PER_TILE

    pltpu.make_async_copy(zsum.at[pl.ds(rbase, _ROWS_PER_TILE)],
                          acc.at[pl.ds(rbase, _ROWS_PER_TILE)], sem_ld).start()
    pltpu.make_async_copy(zcnt.at[pl.ds(rbase, _ROWS_PER_TILE)],
                          acc_cnt.at[pl.ds(rbase, _ROWS_PER_TILE)],
                          sem_ld).start()

    def ones_row(i, carry):
        obuf[i, :] = jnp.ones((_CW,), jnp.float32)
        return carry
    lax.fori_loop(0, _CH, ones_row, 0)

    cmbb = (cmb0, cmb1)
    gidxr = (gidx0, gidx1)
    bufs = (gbuf0, gbuf1)
    rsems = (semr0, semr1)
    gsems = (semg0, semg1)
    ssems = (sems0, sems1)

    def rows(p, b):
        return pltpu.make_async_copy(cmb.at[pl.ds(p * _K, _K)], cmbb[b],
                                     rsems[b])

    def gathers(b):
        return [pltpu.make_async_copy(table.at[gidxr[b].at[k]],
                                      bufs[b].at[pl.ds(k * _CH, _CH)],
                                      gsems[b])
                for k in range(_K)]

    def scats(b):
        return [pltpu.make_async_copy(bufs[b].at[pl.ds(k * _CH, _CH)],
                                      acc.at[cmbb[b].at[k, 2]], ssems[b])
                for k in range(_K)]

    def scat_ones(b):
        return [pltpu.make_async_copy(obuf, acc_cnt.at[cmbb[b].at[k, 2]], semo)
                for k in range(_K)]

    def make_gidx(b):
        for k in range(_K):
            for g in range(_CH // 16):
                sl = pl.ds(16 * g, 16)
                gidxr[b][k, sl] = cmbb[b][k, 1, sl] * _N + cmbb[b][k, 0, sl]

    pltpu.make_async_copy(zsum.at[pl.ds(rbase, _ROWS_PER_TILE)],
                          acc.at[pl.ds(rbase, _ROWS_PER_TILE)], sem_ld).wait()
    pltpu.make_async_copy(zcnt.at[pl.ds(rbase, _ROWS_PER_TILE)],
                          acc_cnt.at[pl.ds(rbase, _ROWS_PER_TILE)],
                          sem_ld).wait()
    plsc.subcore_barrier()

    @pl.when(mycnt > 0)
    def _():
        rows(start, 0).start()
        rows(start, 0).wait()
        make_gidx(0)
        for d in gathers(0):
            d.start()

    def chunk_body(j, carry):
        def arm(b):
            ob = 1 - b

            @pl.when(j >= 1)
            def _():
                for d in scats(ob):
                    d.wait()
                for d in scat_ones(ob):
                    d.wait()

            @pl.when(j + 1 < mycnt)
            def _():
                rows(start + j + 1, ob).start()

            for d in gathers(b):
                d.wait()

            @pl.when(j + 1 < mycnt)
            def _():
                rows(start + j + 1, ob).wait()
                make_gidx(ob)
                for d in gathers(ob):
                    d.start()

            for k in range(_K):
                def egroup(q, c2, k=k):
                    nv = plsc.bitcast(cmbb[b][k, 3, pl.ds(16 * q, 16)],
                                      jnp.float32)
                    for i in range(16):
                        nrm = nv[i]
                        e = k * _CH + q * 16 + i
                        for g in range(_H1 // 16):
                            sl = pl.ds(16 * g, 16)
                            bufs[b][e, sl] = bufs[b][e, sl] * nrm
                    return c2
                lax.fori_loop(0, _CH // 16, egroup, 0)

            for d in scats(b):
                d.start(add=True)
            for d in scat_ones(b):
                d.start(add=True)

        @pl.when(j % 2 == 0)
        def _():
            arm(0)

        @pl.when(j % 2 == 1)
        def _():
            arm(1)

        return carry

    lax.fori_loop(0, mycnt, chunk_body, 0)

    @pl.when(mycnt > 0)
    def _():
        @pl.when((mycnt - 1) % 2 == 0)
        def _():
            for d in scats(0):
                d.wait()
            for d in scat_ones(0):
                d.wait()

        @pl.when((mycnt - 1) % 2 == 1)
        def _():
            for d in scats(1):
                d.wait()
            for d in scat_ones(1):
                d.wait()

    plsc.subcore_barrier()
    pltpu.sync_copy(acc.at[pl.ds(rbase, _ROWS_PER_TILE)],
                    outs.at[c, pl.ds(rbase, _ROWS_PER_TILE)])
    pltpu.sync_copy(acc_cnt.at[pl.ds(rbase, _ROWS_PER_TILE)],
                    outc.at[c, pl.ds(rbase, _ROWS_PER_TILE)])


_sc_pass1 = functools.partial(
    pl.kernel,
    out_type=(
        jax.ShapeDtypeStruct((_NC, _N, _H1), jnp.float32),
        jax.ShapeDtypeStruct((_NC, _N, _CW), jnp.float32),
    ),
    mesh=_mesh,
    compiler_params=_sc_params,
    scratch_types=[
        pltpu.VMEM_SHARED((_N, _H1), jnp.float32),
        pltpu.VMEM_SHARED((_N, _CW), jnp.float32),
        pltpu.VMEM((_K, 4, _CH), jnp.int32),
        pltpu.VMEM((_K, 4, _CH), jnp.int32),
        pltpu.VMEM((_K, _CH), jnp.int32),
        pltpu.VMEM((_K, _CH), jnp.int32),
        pltpu.VMEM((_K * _CH, _H1), jnp.float32),
        pltpu.VMEM((_K * _CH, _H1), jnp.float32),
        pltpu.VMEM((_CH, _CW), jnp.float32),
        pltpu.SemaphoreType.DMA,
        pltpu.SemaphoreType.DMA,
        pltpu.SemaphoreType.DMA,
        pltpu.SemaphoreType.DMA,
        pltpu.SemaphoreType.DMA,
        pltpu.SemaphoreType.DMA,
        pltpu.SemaphoreType.DMA,
        pltpu.SemaphoreType.DMA,
    ],
)(_sc1_body)


def _ring_pass(body_scale, use_gidx):
    def _body(table, cmb, zsum, out,
              acc, cmb0, cmb1, cmb2,
              semr0, semr1, semr2, semg0, semg1, semg2,
              sems0, sems1, sems2, sem_ld):
        def _scoped(gbuf0, gbuf1, gbuf2):
            _body_inner(table, cmb, zsum, out, acc, cmb0, cmb1, cmb2,
                        gbuf0, gbuf1, gbuf2,
                        semr0, semr1, semr2, semg0, semg1, semg2,
                        sems0, sems1, sems2, sem_ld)
        pl.run_scoped(_scoped,
                      pltpu.VMEM((_CH, _H1), jnp.float32),
                      pltpu.VMEM((_CH, _H1), jnp.float32),
                      pltpu.VMEM((_CH, _H1), jnp.float32))

    def _body_inner(table, cmb, zsum, out,
                    acc, cmb0, cmb1, cmb2, gbuf0, gbuf1, gbuf2,
                    semr0, semr1, semr2, semg0, semg1, semg2,
                    sems0, sems1, sems2, sem_ld):
        c = lax.axis_index("c")
        s = lax.axis_index("s")
        start, mycnt = _worker_range(c, s)
        rbase = s * _ROWS_PER_TILE

        pltpu.make_async_copy(zsum.at[pl.ds(rbase, _ROWS_PER_TILE)],
                              acc.at[pl.ds(rbase, _ROWS_PER_TILE)],
                              sem_ld).start()

        cmbb = (cmb0, cmb1, cmb2)
        bufs = (gbuf0, gbuf1, gbuf2)
        rsems = (semr0, semr1, semr2)
        gsems = (semg0, semg1, semg2)
        ssems = (sems0, sems1, sems2)

        def rows(p, b):
            return pltpu.make_async_copy(cmb.at[p], cmbb[b], rsems[b])

        def gather(b):
            return pltpu.make_async_copy(table.at[cmbb[b].at[0]], bufs[b],
                                         gsems[b])

        def scat(b):
            return pltpu.make_async_copy(bufs[b], acc.at[cmbb[b].at[2]],
                                         ssems[b])

        def make_gidx(b):
            if not use_gidx:
                return
            for g in range(_CH // 16):
                sl = pl.ds(16 * g, 16)
                cmbb[b][0, sl] = cmbb[b][1, sl] * _N + cmbb[b][0, sl]

        pltpu.make_async_copy(zsum.at[pl.ds(rbase, _ROWS_PER_TILE)],
                              acc.at[pl.ds(rbase, _ROWS_PER_TILE)],
                              sem_ld).wait()
        plsc.subcore_barrier()

        @pl.when(mycnt > 0)
        def _():
            rows(start, 0).start()

            @pl.when(mycnt > 1)
            def _():
                rows(start + 1, 1).start()

            rows(start, 0).wait()
            make_gidx(0)
            gather(0).start()

            @pl.when(mycnt > 1)
            def _():
                rows(start + 1, 1).wait()
                make_gidx(1)
                gather(1).start()

        def chunk_body(j, carry):
            def arm(b):
                nb = (b + 2) % _NBUF

                @pl.when(j >= 1)
                def _():
                    scat(nb).wait()

                @pl.when(j + 2 < mycnt)
                def _():
                    rows(start + j + 2, nb).start()

                gather(b).wait()

                @pl.when(j + 2 < mycnt)
                def _():
                    rows(start + j + 2, nb).wait()
                    make_gidx(nb)
                    gather(nb).start()

                body_scale(bufs, cmbb, b)

                scat(b).start(add=True)

            @pl.when(j % _NBUF == 0)
            def _():
                arm(0)

            @pl.when(j % _NBUF == 1)
            def _():
                arm(1)

            @pl.when(j % _NBUF == 2)
            def _():
                arm(2)

            return carry

        lax.fori_loop(0, mycnt, chunk_body, 0)

        @pl.when(mycnt > 0)
        def _():
            for b in range(_NBUF):
                @pl.when((mycnt - 1) % _NBUF == b)
                def _(b=b):
                    scat(b).wait()

        plsc.subcore_barrier()
        pltpu.sync_copy(acc.at[pl.ds(rbase, _ROWS_PER_TILE)],
                        out.at[c, pl.ds(rbase, _ROWS_PER_TILE)])

    return _body


def _no_scale(bufs, cmbb, b):
    pass


_ring_scratch = [
    pltpu.VMEM_SHARED((_N, _H1), jnp.float32),
    pltpu.VMEM((4, _CH), jnp.int32),
    pltpu.VMEM((4, _CH), jnp.int32),
    pltpu.VMEM((4, _CH), jnp.int32),
    pltpu.SemaphoreType.DMA,
    pltpu.SemaphoreType.DMA,
    pltpu.SemaphoreType.DMA,
    pltpu.SemaphoreType.DMA,
    pltpu.SemaphoreType.DMA,
    pltpu.SemaphoreType.DMA,
    pltpu.SemaphoreType.DMA,
    pltpu.SemaphoreType.DMA,
    pltpu.SemaphoreType.DMA,
    pltpu.SemaphoreType.DMA,
]

_sc_pass2 = functools.partial(
    pl.kernel,
    out_type=jax.ShapeDtypeStruct((_NC, _N, _H2), jnp.float32),
    mesh=_mesh,
    compiler_params=_sc_params,
    scratch_types=list(_ring_scratch),
)(_ring_pass(_no_scale, use_gidx=False))


def _dense_c_body(p_ref, c_ref, xr_ref, wn_ref, wl_ref, b2_ref, h_ref, y2_ref):
    sums = p_ref[0] + p_ref[1]
    cnt = c_ref[0] + c_ref[1]
    cnt0 = jnp.max(cnt, axis=1, keepdims=True)
    x1 = sums / jnp.maximum(cnt0, 1.0) + xr_ref[...]
    h_ref[...] = jnp.dot(x1, wn_ref[...], preferred_element_type=jnp.float32)
    y2_ref[...] = (jnp.dot(x1, wl_ref[...], preferred_element_type=jnp.float32)
                   + b2_ref[...])


_dense_c = pl.pallas_call(
    _dense_c_body,
    out_shape=[
        jax.ShapeDtypeStruct((_N, _H1), jnp.float32),
        jax.ShapeDtypeStruct((_N, _H2), jnp.float32),
    ],
)


def _dense_e_body(q_ref, y2_ref, o_ref):
    o_ref[...] = q_ref[0] + q_ref[1] + y2_ref[...]


_dense_e = pl.pallas_call(
    _dense_e_body,
    out_shape=jax.ShapeDtypeStruct((_N, _H2), jnp.float32),
)


def kernel(node_features, edge_index, edge_type, edge_norm, basis, att, root,
           bias1, w_nbr, w_lin, bias2):
    src2 = edge_index[0].reshape(_NCHUNK, 1, _CH)
    typ2 = edge_type.reshape(_NCHUNK, 1, _CH)
    dst2 = edge_index[1].reshape(_NCHUNK, 1, _CH)
    norm_bits = lax.bitcast_convert_type(edge_norm, jnp.int32)
    norm2 = norm_bits.reshape(_NCHUNK, 1, _CH)
    cmb = jnp.concatenate([src2, typ2, dst2, norm2], axis=1)
    zsum = jnp.zeros((_N, _H1), jnp.float32)
    zcnt = jnp.zeros((_N, _CW), jnp.float32)

    xw, xr = _dense_a(att, basis, node_features, root, bias1.reshape(1, _H1))
    xw_flat = xw.reshape(_R * _N, _H1)
    part1, cnt1 = _sc_pass1(xw_flat, cmb, zsum, zcnt)
    h, y2 = _dense_c(part1, cnt1, xr, w_nbr, w_lin, bias2.reshape(1, _H2))
    part2 = _sc_pass2(h, cmb, zsum)
    return _dense_e(part2, y2)


# speedup vs baseline: 1.0490x; 1.0115x over previous
import functools

import jax
import jax.numpy as jnp
from jax import lax
from jax.experimental import pallas as pl
from jax.experimental.pallas import tpu as pltpu
from jax.experimental.pallas import tpu_sc as plsc

_N = 10000
_E = 320000
_D = 128
_H1 = 128
_H2 = 128
_R = 8
_NB = 30

_NC = 2
_NS = 16
_NW = _NC * _NS
_CH = 128
_NCHUNK = _E // _CH
_Q, _REM = divmod(_NCHUNK, _NW)
_ROWS_PER_TILE = _N // _NS
_CW = 16
_NBUF = 3
_K = 1

_mesh = plsc.VectorSubcoreMesh(
    core_axis_name="c", subcore_axis_name="s", num_cores=_NC, num_subcores=_NS)
_sc_params = pltpu.CompilerParams(use_tc_tiling_on_sc=False,
                                  needs_layout_passes=False)


def _worker_range(c, s):
    w = s * _NC + c
    start = w * _Q + jnp.minimum(w, _REM)
    mycnt = _Q + (w < _REM).astype(jnp.int32)
    return start, mycnt


def _dense_a_body(att_ref, basis_ref, x_ref, root_ref, b1_ref, xw_ref, xr_ref):
    r = pl.program_id(0)

    def bstep(b, acc):
        return acc + att_ref[r, b] * basis_ref[b]

    wr = lax.fori_loop(0, _NB, bstep, jnp.zeros((_D, _H1), jnp.float32))
    xw_ref[0] = jnp.dot(x_ref[...], wr, preferred_element_type=jnp.float32)

    @pl.when(r == 0)
    def _():
        xr_ref[...] = (jnp.dot(x_ref[...], root_ref[...],
                               preferred_element_type=jnp.float32)
                       + b1_ref[...])


_dense_a = pl.pallas_call(
    _dense_a_body,
    grid=(_R,),
    in_specs=[
        pl.BlockSpec(memory_space=pltpu.SMEM),
        pl.BlockSpec((_NB, _D, _H1), lambda r: (0, 0, 0)),
        pl.BlockSpec((_N, _D), lambda r: (0, 0)),
        pl.BlockSpec((_D, _H1), lambda r: (0, 0)),
        pl.BlockSpec((1, _H1), lambda r: (0, 0)),
    ],
    out_specs=[
        pl.BlockSpec((1, _N, _H1), lambda r: (r, 0, 0)),
        pl.BlockSpec((_N, _H1), lambda r: (0, 0)),
    ],
    out_shape=[
        jax.ShapeDtypeStruct((_R, _N, _H1), jnp.float32),
        jax.ShapeDtypeStruct((_N, _H1), jnp.float32),
    ],
)


def _sc1_body(table, src2, typ2, dst2, norm2, zsum, zcnt, outs, outc,
              acc, acc_cnt,
              srcr0, srcr1, typr0, typr1, dstr0, dstr1, normr0, normr1,
              gidx0, gidx1, gbuf0, gbuf1, obuf,
              semr0, semr1, semg0, semg1, sems0, sems1, semo, sem_ld):
    c = lax.axis_index("c")
    s = lax.axis_index("s")
    start, mycnt = _worker_range(c, s)
    rbase = s * _ROWS_PER_TILE

    pltpu.make_async_copy(zsum.at[pl.ds(rbase, _ROWS_PER_TILE)],
                          acc.at[pl.ds(rbase, _ROWS_PER_TILE)], sem_ld).start()
    pltpu.make_async_copy(zcnt.at[pl.ds(rbase, _ROWS_PER_TILE)],
                          acc_cnt.at[pl.ds(rbase, _ROWS_PER_TILE)],
                          sem_ld).start()

    def ones_row(i, carry):
        obuf[i, :] = jnp.ones((_CW,), jnp.float32)
        return carry
    lax.fori_loop(0, _CH, ones_row, 0)

    srcr = (srcr0, srcr1)
    typr = (typr0, typr1)
    dstr = (dstr0, dstr1)
    normr = (normr0, normr1)
    gidxr = (gidx0, gidx1)
    bufs = (gbuf0, gbuf1)
    rsems = (semr0, semr1)
    gsems = (semg0, semg1)
    ssems = (sems0, sems1)

    class _Rows:
        def __init__(self, p, b):
            self.ds = (
                pltpu.make_async_copy(src2.at[p], srcr[b], rsems[b]),
                pltpu.make_async_copy(typ2.at[p], typr[b], rsems[b]),
                pltpu.make_async_copy(dst2.at[p], dstr[b], rsems[b]),
                pltpu.make_async_copy(norm2.at[p], normr[b], rsems[b]),
            )

        def start(self):
            for d in self.ds:
                d.start()

        def wait(self):
            for d in self.ds:
                d.wait()

    rows = _Rows

    def gathers(b):
        return [pltpu.make_async_copy(table.at[gidxr[b]], bufs[b], gsems[b])]

    def scats(b):
        return [pltpu.make_async_copy(bufs[b], acc.at[dstr[b]], ssems[b])]

    def scat_ones(b):
        return [pltpu.make_async_copy(obuf, acc_cnt.at[dstr[b]], semo)]

    def make_gidx(b):
        for g in range(_CH // 16):
            sl = pl.ds(16 * g, 16)
            gidxr[b][sl] = typr[b][sl] * _N + srcr[b][sl]

    pltpu.make_async_copy(zsum.at[pl.ds(rbase, _ROWS_PER_TILE)],
                          acc.at[pl.ds(rbase, _ROWS_PER_TILE)], sem_ld).wait()
    pltpu.make_async_copy(zcnt.at[pl.ds(rbase, _ROWS_PER_TILE)],
                          acc_cnt.at[pl.ds(rbase, _ROWS_PER_TILE)],
                          sem_ld).wait()
    plsc.subcore_barrier()

    @pl.when(mycnt > 0)
    def _():
        rows(start, 0).start()
        rows(start, 0).wait()
        make_gidx(0)
        for d in gathers(0):
            d.start()

    def chunk_body(j, carry):
        def arm(b):
            ob = 1 - b

            @pl.when(j >= 1)
            def _():
                for d in scats(ob):
                    d.wait()
                for d in scat_ones(ob):
                    d.wait()

            @pl.when(j + 1 < mycnt)
            def _():
                rows(start + j + 1, ob).start()

            for d in gathers(b):
                d.wait()

            @pl.when(j + 1 < mycnt)
            def _():
                rows(start + j + 1, ob).wait()
                make_gidx(ob)
                for d in gathers(ob):
                    d.start()

            def egroup(q, c2):
                nv = normr[b][pl.ds(16 * q, 16)]
                for i in range(16):
                    nrm = nv[i]
                    e = q * 16 + i
                    for g in range(_H1 // 16):
                        sl = pl.ds(16 * g, 16)
                        bufs[b][e, sl] = bufs[b][e, sl] * nrm
                return c2
            lax.fori_loop(0, _CH // 16, egroup, 0)

            for d in scats(b):
                d.start(add=True)
            for d in scat_ones(b):
                d.start(add=True)

        @pl.when(j % 2 == 0)
        def _():
            arm(0)

        @pl.when(j % 2 == 1)
        def _():
            arm(1)

        return carry

    lax.fori_loop(0, mycnt, chunk_body, 0)

    @pl.when(mycnt > 0)
    def _():
        @pl.when((mycnt - 1) % 2 == 0)
        def _():
            for d in scats(0):
                d.wait()
            for d in scat_ones(0):
                d.wait()

        @pl.when((mycnt - 1) % 2 == 1)
        def _():
            for d in scats(1):
                d.wait()
            for d in scat_ones(1):
                d.wait()

    plsc.subcore_barrier()
    pltpu.sync_copy(acc.at[pl.ds(rbase, _ROWS_PER_TILE)],
                    outs.at[c, pl.ds(rbase, _ROWS_PER_TILE)])
    pltpu.sync_copy(acc_cnt.at[pl.ds(rbase, _ROWS_PER_TILE)],
                    outc.at[c, pl.ds(rbase, _ROWS_PER_TILE)])


_sc_pass1 = functools.partial(
    pl.kernel,
    out_type=(
        jax.ShapeDtypeStruct((_NC, _N, _H1), jnp.float32),
        jax.ShapeDtypeStruct((_NC, _N, _CW), jnp.float32),
    ),
    mesh=_mesh,
    compiler_params=_sc_params,
    scratch_types=[
        pltpu.VMEM_SHARED((_N, _H1), jnp.float32),
        pltpu.VMEM_SHARED((_N, _CW), jnp.float32),
        pltpu.VMEM((_CH,), jnp.int32),
        pltpu.VMEM((_CH,), jnp.int32),
        pltpu.VMEM((_CH,), jnp.int32),
        pltpu.VMEM((_CH,), jnp.int32),
        pltpu.VMEM((_CH,), jnp.int32),
        pltpu.VMEM((_CH,), jnp.int32),
        pltpu.VMEM((_CH,), jnp.float32),
        pltpu.VMEM((_CH,), jnp.float32),
        pltpu.VMEM((_CH,), jnp.int32),
        pltpu.VMEM((_CH,), jnp.int32),
        pltpu.VMEM((_CH, _H1), jnp.float32),
        pltpu.VMEM((_CH, _H1), jnp.float32),
        pltpu.VMEM((_CH, _CW), jnp.float32),
        pltpu.SemaphoreType.DMA,
        pltpu.SemaphoreType.DMA,
        pltpu.SemaphoreType.DMA,
        pltpu.SemaphoreType.DMA,
        pltpu.SemaphoreType.DMA,
        pltpu.SemaphoreType.DMA,
        pltpu.SemaphoreType.DMA,
        pltpu.SemaphoreType.DMA,
    ],
)(_sc1_body)


def _ring_pass(body_scale, use_gidx):
    def _body(table, src2, dst2, zsum, out,
              acc, srcb0, srcb1, srcb2, dstb0, dstb1, dstb2,
              semr0, semr1, semr2, semg0, semg1, semg2,
              sems0, sems1, sems2, sem_ld):
        def _scoped(gbuf0, gbuf1, gbuf2):
            _body_inner(table, src2, dst2, zsum, out, acc,
                        srcb0, srcb1, srcb2, dstb0, dstb1, dstb2,
                        gbuf0, gbuf1, gbuf2,
                        semr0, semr1, semr2, semg0, semg1, semg2,
                        sems0, sems1, sems2, sem_ld)
        pl.run_scoped(_scoped,
                      pltpu.VMEM((_CH, _H1), jnp.float32),
                      pltpu.VMEM((_CH, _H1), jnp.float32),
                      pltpu.VMEM((_CH, _H1), jnp.float32))

    def _body_inner(table, src2, dst2, zsum, out,
                    acc, srcb0, srcb1, srcb2, dstb0, dstb1, dstb2,
                    gbuf0, gbuf1, gbuf2,
                    semr0, semr1, semr2, semg0, semg1, semg2,
                    sems0, sems1, sems2, sem_ld):
        c = lax.axis_index("c")
        s = lax.axis_index("s")
        start, mycnt = _worker_range(c, s)
        rbase = s * _ROWS_PER_TILE

        pltpu.make_async_copy(zsum.at[pl.ds(rbase, _ROWS_PER_TILE)],
                              acc.at[pl.ds(rbase, _ROWS_PER_TILE)],
                              sem_ld).start()

        srcb = (srcb0, srcb1, srcb2)
        dstb = (dstb0, dstb1, dstb2)
        bufs = (gbuf0, gbuf1, gbuf2)
        rsems = (semr0, semr1, semr2)
        gsems = (semg0, semg1, semg2)
        ssems = (sems0, sems1, sems2)

        class _Rows:
            def __init__(self, p, b):
                self.ds = (
                    pltpu.make_async_copy(src2.at[p], srcb[b], rsems[b]),
                    pltpu.make_async_copy(dst2.at[p], dstb[b], rsems[b]),
                )

            def start(self):
                for d in self.ds:
                    d.start()

            def wait(self):
                for d in self.ds:
                    d.wait()

        rows = _Rows

        def gather(b):
            return pltpu.make_async_copy(table.at[srcb[b]], bufs[b], gsems[b])

        def scat(b):
            return pltpu.make_async_copy(bufs[b], acc.at[dstb[b]], ssems[b])

        def make_gidx(b):
            del b

        pltpu.make_async_copy(zsum.at[pl.ds(rbase, _ROWS_PER_TILE)],
                              acc.at[pl.ds(rbase, _ROWS_PER_TILE)],
                              sem_ld).wait()
        plsc.subcore_barrier()

        @pl.when(mycnt > 0)
        def _():
            rows(start, 0).start()

            @pl.when(mycnt > 1)
            def _():
                rows(start + 1, 1).start()

            rows(start, 0).wait()
            make_gidx(0)
            gather(0).start()

            @pl.when(mycnt > 1)
            def _():
                rows(start + 1, 1).wait()
                make_gidx(1)
                gather(1).start()

        def chunk_body(j, carry):
            def arm(b):
                nb = (b + 2) % _NBUF

                @pl.when(j >= 1)
                def _():
                    scat(nb).wait()

                @pl.when(j + 2 < mycnt)
                def _():
                    rows(start + j + 2, nb).start()

                gather(b).wait()

                @pl.when(j + 2 < mycnt)
                def _():
                    rows(start + j + 2, nb).wait()
                    make_gidx(nb)
                    gather(nb).start()

                body_scale(bufs, b)

                scat(b).start(add=True)

            @pl.when(j % _NBUF == 0)
            def _():
                arm(0)

            @pl.when(j % _NBUF == 1)
            def _():
                arm(1)

            @pl.when(j % _NBUF == 2)
            def _():
                arm(2)

            return carry

        lax.fori_loop(0, mycnt, chunk_body, 0)

        @pl.when(mycnt > 0)
        def _():
            for b in range(_NBUF):
                @pl.when((mycnt - 1) % _NBUF == b)
                def _(b=b):
                    scat(b).wait()

        plsc.subcore_barrier()
        pltpu.sync_copy(acc.at[pl.ds(rbase, _ROWS_PER_TILE)],
                        out.at[c, pl.ds(rbase, _ROWS_PER_TILE)])

    return _body


def _no_scale(bufs, b):
    pass


_ring_scratch = [
    pltpu.VMEM_SHARED((_N, _H1), jnp.float32),
    pltpu.VMEM((_CH,), jnp.int32),
    pltpu.VMEM((_CH,), jnp.int32),
    pltpu.VMEM((_CH,), jnp.int32),
    pltpu.VMEM((_CH,), jnp.int32),
    pltpu.VMEM((_CH,), jnp.int32),
    pltpu.VMEM((_CH,), jnp.int32),
    pltpu.SemaphoreType.DMA,
    pltpu.SemaphoreType.DMA,
    pltpu.SemaphoreType.DMA,
    pltpu.SemaphoreType.DMA,
    pltpu.SemaphoreType.DMA,
    pltpu.SemaphoreType.DMA,
    pltpu.SemaphoreType.DMA,
    pltpu.SemaphoreType.DMA,
    pltpu.SemaphoreType.DMA,
    pltpu.SemaphoreType.DMA,
]

_sc_pass2 = functools.partial(
    pl.kernel,
    out_type=jax.ShapeDtypeStruct((_NC, _N, _H2), jnp.float32),
    mesh=_mesh,
    compiler_params=_sc_params,
    scratch_types=list(_ring_scratch),
)(_ring_pass(_no_scale, use_gidx=False))


def _dense_c_body(p_ref, c_ref, xr_ref, wn_ref, wl_ref, b2_ref, h_ref, y2_ref):
    sums = p_ref[0] + p_ref[1]
    cnt = c_ref[0] + c_ref[1]
    cnt0 = jnp.max(cnt, axis=1, keepdims=True)
    x1 = sums / jnp.maximum(cnt0, 1.0) + xr_ref[...]
    h_ref[...] = jnp.dot(x1, wn_ref[...], preferred_element_type=jnp.float32)
    y2_ref[...] = (jnp.dot(x1, wl_ref[...], preferred_element_type=jnp.float32)
                   + b2_ref[...])


_dense_c = pl.pallas_call(
    _dense_c_body,
    out_shape=[
        jax.ShapeDtypeStruct((_N, _H1), jnp.float32),
        jax.ShapeDtypeStruct((_N, _H2), jnp.float32),
    ],
)


def _dense_e_body(q_ref, y2_ref, o_ref):
    o_ref[...] = q_ref[0] + q_ref[1] + y2_ref[...]


_dense_e = pl.pallas_call(
    _dense_e_body,
    out_shape=jax.ShapeDtypeStruct((_N, _H2), jnp.float32),
)


def kernel(node_features, edge_index, edge_type, edge_norm, basis, att, root,
           bias1, w_nbr, w_lin, bias2):
    src2 = edge_index[0].reshape(_NCHUNK, _CH)
    typ2 = edge_type.reshape(_NCHUNK, _CH)
    dst2 = edge_index[1].reshape(_NCHUNK, _CH)
    norm2 = edge_norm.reshape(_NCHUNK, _CH)
    zsum = jnp.zeros((_N, _H1), jnp.float32)
    zcnt = jnp.zeros((_N, _CW), jnp.float32)

    xw, xr = _dense_a(att, basis, node_features, root, bias1.reshape(1, _H1))
    xw_flat = xw.reshape(_R * _N, _H1)
    part1, cnt1 = _sc_pass1(xw_flat, src2, typ2, dst2, norm2, zsum, zcnt)
    h, y2 = _dense_c(part1, cnt1, xr, w_nbr, w_lin, bias2.reshape(1, _H2))
    part2 = _sc_pass2(h, src2, dst2, zsum)
    return _dense_e(part2, y2)


# speedup vs baseline: 1.0506x; 1.0016x over previous
import functools

import jax
import jax.numpy as jnp
from jax import lax
from jax.experimental import pallas as pl
from jax.experimental.pallas import tpu as pltpu
from jax.experimental.pallas import tpu_sc as plsc

_N = 10000
_E = 320000
_D = 128
_H1 = 128
_H2 = 128
_R = 8
_NB = 30

_NC = 2
_NS = 16
_NW = _NC * _NS
_CH = 128
_NCHUNK = _E // _CH
_Q, _REM = divmod(_NCHUNK, _NW)
_ROWS_PER_TILE = _N // _NS
_CW = 16
_NBUF = 3
_K = 1

_mesh = plsc.VectorSubcoreMesh(
    core_axis_name="c", subcore_axis_name="s", num_cores=_NC, num_subcores=_NS)
_sc_params = pltpu.CompilerParams(use_tc_tiling_on_sc=False,
                                  needs_layout_passes=False)


def _worker_range(c, s):
    w = s * _NC + c
    start = w * _Q + jnp.minimum(w, _REM)
    mycnt = _Q + (w < _REM).astype(jnp.int32)
    return start, mycnt


def _dense_a_body(att_ref, basis_ref, x_ref, root_ref, b1_ref, xw_ref, xr_ref):
    r = pl.program_id(0)

    def bstep(b, acc):
        return acc + att_ref[r, b] * basis_ref[b]

    wr = lax.fori_loop(0, _NB, bstep, jnp.zeros((_D, _H1), jnp.float32))
    xw_ref[0] = jnp.dot(x_ref[...], wr, preferred_element_type=jnp.float32)

    @pl.when(r == 0)
    def _():
        xr_ref[...] = (jnp.dot(x_ref[...], root_ref[...],
                               preferred_element_type=jnp.float32)
                       + b1_ref[...])


_dense_a = pl.pallas_call(
    _dense_a_body,
    grid=(_R,),
    in_specs=[
        pl.BlockSpec(memory_space=pltpu.SMEM),
        pl.BlockSpec((_NB, _D, _H1), lambda r: (0, 0, 0)),
        pl.BlockSpec((_N, _D), lambda r: (0, 0)),
        pl.BlockSpec((_D, _H1), lambda r: (0, 0)),
        pl.BlockSpec((1, _H1), lambda r: (0, 0)),
    ],
    out_specs=[
        pl.BlockSpec((1, _N, _H1), lambda r: (r, 0, 0)),
        pl.BlockSpec((_N, _H1), lambda r: (0, 0)),
    ],
    out_shape=[
        jax.ShapeDtypeStruct((_R, _N, _H1), jnp.float32),
        jax.ShapeDtypeStruct((_N, _H1), jnp.float32),
    ],
)


def _sc1_body(table, src2, typ2, dst2, norm2, zsum, zcnt, outs, outc,
              acc, acc_cnt,
              srcr0, srcr1, typr0, typr1, dstr0, dstr1, normr0, normr1,
              gidx0, gidx1, gbuf0, gbuf1, obuf,
              semr0, semr1, semg0, semg1, sems0, sems1, semo, sem_ld):
    c = lax.axis_index("c")
    s = lax.axis_index("s")
    start, mycnt = _worker_range(c, s)
    rbase = s * _ROWS_PER_TILE

    pltpu.make_async_copy(zsum.at[pl.ds(0, _ROWS_PER_TILE)],
                          acc.at[pl.ds(rbase, _ROWS_PER_TILE)], sem_ld).start()
    pltpu.make_async_copy(zcnt.at[pl.ds(0, _ROWS_PER_TILE)],
                          acc_cnt.at[pl.ds(rbase, _ROWS_PER_TILE)],
                          sem_ld).start()

    def ones_row(i, carry):
        obuf[i, :] = jnp.ones((_CW,), jnp.float32)
        return carry
    lax.fori_loop(0, _CH, ones_row, 0)

    srcr = (srcr0, srcr1)
    typr = (typr0, typr1)
    dstr = (dstr0, dstr1)
    normr = (normr0, normr1)
    gidxr = (gidx0, gidx1)
    bufs = (gbuf0, gbuf1)
    rsems = (semr0, semr1)
    gsems = (semg0, semg1)
    ssems = (sems0, sems1)

    class _Rows:
        def __init__(self, p, b):
            self.ds = (
                pltpu.make_async_copy(src2.at[p], srcr[b], rsems[b]),
                pltpu.make_async_copy(typ2.at[p], typr[b], rsems[b]),
                pltpu.make_async_copy(dst2.at[p], dstr[b], rsems[b]),
                pltpu.make_async_copy(norm2.at[p], normr[b], rsems[b]),
            )

        def start(self):
            for d in self.ds:
                d.start()

        def wait(self):
            for d in self.ds:
                d.wait()

    rows = _Rows

    def gathers(b):
        return [pltpu.make_async_copy(table.at[gidxr[b]], bufs[b], gsems[b])]

    def scats(b):
        return [pltpu.make_async_copy(bufs[b], acc.at[dstr[b]], ssems[b])]

    def scat_ones(b):
        return [pltpu.make_async_copy(obuf, acc_cnt.at[dstr[b]], semo)]

    def make_gidx(b):
        for g in range(_CH // 16):
            sl = pl.ds(16 * g, 16)
            gidxr[b][sl] = typr[b][sl] * _N + srcr[b][sl]

    @pl.when(mycnt > 0)
    def _():
        rows(start, 0).start()
        rows(start, 0).wait()
        make_gidx(0)
        for d in gathers(0):
            d.start()

    pltpu.make_async_copy(zsum.at[pl.ds(0, _ROWS_PER_TILE)],
                          acc.at[pl.ds(rbase, _ROWS_PER_TILE)], sem_ld).wait()
    pltpu.make_async_copy(zcnt.at[pl.ds(0, _ROWS_PER_TILE)],
                          acc_cnt.at[pl.ds(rbase, _ROWS_PER_TILE)],
                          sem_ld).wait()
    plsc.subcore_barrier()

    def chunk_body(j, carry):
        def arm(b):
            ob = 1 - b

            @pl.when(j >= 1)
            def _():
                for d in scats(ob):
                    d.wait()
                for d in scat_ones(ob):
                    d.wait()

            @pl.when(j + 1 < mycnt)
            def _():
                rows(start + j + 1, ob).start()

            for d in gathers(b):
                d.wait()

            @pl.when(j + 1 < mycnt)
            def _():
                rows(start + j + 1, ob).wait()
                make_gidx(ob)
                for d in gathers(ob):
                    d.start()

            def egroup(q, c2):
                nv = normr[b][pl.ds(16 * q, 16)]
                for i in range(16):
                    nrm = nv[i]
                    e = q * 16 + i
                    for g in range(_H1 // 16):
                        sl = pl.ds(16 * g, 16)
                        bufs[b][e, sl] = bufs[b][e, sl] * nrm
                return c2
            lax.fori_loop(0, _CH // 16, egroup, 0)

            for d in scats(b):
                d.start(add=True)
            for d in scat_ones(b):
                d.start(add=True)

        @pl.when(j % 2 == 0)
        def _():
            arm(0)

        @pl.when(j % 2 == 1)
        def _():
            arm(1)

        return carry

    lax.fori_loop(0, mycnt, chunk_body, 0)

    @pl.when(mycnt > 0)
    def _():
        @pl.when((mycnt - 1) % 2 == 0)
        def _():
            for d in scats(0):
                d.wait()
            for d in scat_ones(0):
                d.wait()

        @pl.when((mycnt - 1) % 2 == 1)
        def _():
            for d in scats(1):
                d.wait()
            for d in scat_ones(1):
                d.wait()

    plsc.subcore_barrier()
    pltpu.sync_copy(acc.at[pl.ds(rbase, _ROWS_PER_TILE)],
                    outs.at[c, pl.ds(rbase, _ROWS_PER_TILE)])
    pltpu.sync_copy(acc_cnt.at[pl.ds(rbase, _ROWS_PER_TILE)],
                    outc.at[c, pl.ds(rbase, _ROWS_PER_TILE)])


_sc_pass1 = functools.partial(
    pl.kernel,
    out_type=(
        jax.ShapeDtypeStruct((_NC, _N, _H1), jnp.float32),
        jax.ShapeDtypeStruct((_NC, _N, _CW), jnp.float32),
    ),
    mesh=_mesh,
    compiler_params=_sc_params,
    scratch_types=[
        pltpu.VMEM_SHARED((_N, _H1), jnp.float32),
        pltpu.VMEM_SHARED((_N, _CW), jnp.float32),
        pltpu.VMEM((_CH,), jnp.int32),
        pltpu.VMEM((_CH,), jnp.int32),
        pltpu.VMEM((_CH,), jnp.int32),
        pltpu.VMEM((_CH,), jnp.int32),
        pltpu.VMEM((_CH,), jnp.int32),
        pltpu.VMEM((_CH,), jnp.int32),
        pltpu.VMEM((_CH,), jnp.float32),
        pltpu.VMEM((_CH,), jnp.float32),
        pltpu.VMEM((_CH,), jnp.int32),
        pltpu.VMEM((_CH,), jnp.int32),
        pltpu.VMEM((_CH, _H1), jnp.float32),
        pltpu.VMEM((_CH, _H1), jnp.float32),
        pltpu.VMEM((_CH, _CW), jnp.float32),
        pltpu.SemaphoreType.DMA,
        pltpu.SemaphoreType.DMA,
        pltpu.SemaphoreType.DMA,
        pltpu.SemaphoreType.DMA,
        pltpu.SemaphoreType.DMA,
        pltpu.SemaphoreType.DMA,
        pltpu.SemaphoreType.DMA,
        pltpu.SemaphoreType.DMA,
    ],
)(_sc1_body)


def _ring_pass(body_scale, use_gidx):
    def _body(table, src2, dst2, zsum, out,
              acc, srcb0, srcb1, srcb2, dstb0, dstb1, dstb2,
              semr0, semr1, semr2, semg0, semg1, semg2,
              sems0, sems1, sems2, sem_ld):
        def _scoped(gbuf0, gbuf1, gbuf2):
            _body_inner(table, src2, dst2, zsum, out, acc,
                        srcb0, srcb1, srcb2, dstb0, dstb1, dstb2,
                        gbuf0, gbuf1, gbuf2,
                        semr0, semr1, semr2, semg0, semg1, semg2,
                        sems0, sems1, sems2, sem_ld)
        pl.run_scoped(_scoped,
                      pltpu.VMEM((_CH, _H1), jnp.float32),
                      pltpu.VMEM((_CH, _H1), jnp.float32),
                      pltpu.VMEM((_CH, _H1), jnp.float32))

    def _body_inner(table, src2, dst2, zsum, out,
                    acc, srcb0, srcb1, srcb2, dstb0, dstb1, dstb2,
                    gbuf0, gbuf1, gbuf2,
                    semr0, semr1, semr2, semg0, semg1, semg2,
                    sems0, sems1, sems2, sem_ld):
        c = lax.axis_index("c")
        s = lax.axis_index("s")
        start, mycnt = _worker_range(c, s)
        rbase = s * _ROWS_PER_TILE

        pltpu.make_async_copy(zsum.at[pl.ds(0, _ROWS_PER_TILE)],
                              acc.at[pl.ds(rbase, _ROWS_PER_TILE)],
                              sem_ld).start()

        srcb = (srcb0, srcb1, srcb2)
        dstb = (dstb0, dstb1, dstb2)
        bufs = (gbuf0, gbuf1, gbuf2)
        rsems = (semr0, semr1, semr2)
        gsems = (semg0, semg1, semg2)
        ssems = (sems0, sems1, sems2)

        class _Rows:
            def __init__(self, p, b):
                self.ds = (
                    pltpu.make_async_copy(src2.at[p], srcb[b], rsems[b]),
                    pltpu.make_async_copy(dst2.at[p], dstb[b], rsems[b]),
                )

            def start(self):
                for d in self.ds:
                    d.start()

            def wait(self):
                for d in self.ds:
                    d.wait()

        rows = _Rows

        def gather(b):
            return pltpu.make_async_copy(table.at[srcb[b]], bufs[b], gsems[b])

        def scat(b):
            return pltpu.make_async_copy(bufs[b], acc.at[dstb[b]], ssems[b])

        def make_gidx(b):
            del b

        @pl.when(mycnt > 0)
        def _():
            rows(start, 0).start()

            @pl.when(mycnt > 1)
            def _():
                rows(start + 1, 1).start()

            rows(start, 0).wait()
            make_gidx(0)
            gather(0).start()

            @pl.when(mycnt > 1)
            def _():
                rows(start + 1, 1).wait()
                make_gidx(1)
                gather(1).start()

        pltpu.make_async_copy(zsum.at[pl.ds(0, _ROWS_PER_TILE)],
                              acc.at[pl.ds(rbase, _ROWS_PER_TILE)],
                              sem_ld).wait()
        plsc.subcore_barrier()

        def chunk_body(j, carry):
            def arm(b):
                nb = (b + 2) % _NBUF

                @pl.when(j >= 1)
                def _():
                    scat(nb).wait()

                @pl.when(j + 2 < mycnt)
                def _():
                    rows(start + j + 2, nb).start()

                gather(b).wait()

                @pl.when(j + 2 < mycnt)
                def _():
                    rows(start + j + 2, nb).wait()
                    make_gidx(nb)
                    gather(nb).start()

                body_scale(bufs, b)

                scat(b).start(add=True)

            @pl.when(j % _NBUF == 0)
            def _():
                arm(0)

            @pl.when(j % _NBUF == 1)
            def _():
                arm(1)

            @pl.when(j % _NBUF == 2)
            def _():
                arm(2)

            return carry

        lax.fori_loop(0, mycnt, chunk_body, 0)

        @pl.when(mycnt > 0)
        def _():
            for b in range(_NBUF):
                @pl.when((mycnt - 1) % _NBUF == b)
                def _(b=b):
                    scat(b).wait()

        plsc.subcore_barrier()
        pltpu.sync_copy(acc.at[pl.ds(rbase, _ROWS_PER_TILE)],
                        out.at[c, pl.ds(rbase, _ROWS_PER_TILE)])

    return _body


def _no_scale(bufs, b):
    pass


_ring_scratch = [
    pltpu.VMEM_SHARED((_N, _H1), jnp.float32),
    pltpu.VMEM((_CH,), jnp.int32),
    pltpu.VMEM((_CH,), jnp.int32),
    pltpu.VMEM((_CH,), jnp.int32),
    pltpu.VMEM((_CH,), jnp.int32),
    pltpu.VMEM((_CH,), jnp.int32),
    pltpu.VMEM((_CH,), jnp.int32),
    pltpu.SemaphoreType.DMA,
    pltpu.SemaphoreType.DMA,
    pltpu.SemaphoreType.DMA,
    pltpu.SemaphoreType.DMA,
    pltpu.SemaphoreType.DMA,
    pltpu.SemaphoreType.DMA,
    pltpu.SemaphoreType.DMA,
    pltpu.SemaphoreType.DMA,
    pltpu.SemaphoreType.DMA,
    pltpu.SemaphoreType.DMA,
]

_sc_pass2 = functools.partial(
    pl.kernel,
    out_type=jax.ShapeDtypeStruct((_NC, _N, _H2), jnp.float32),
    mesh=_mesh,
    compiler_params=_sc_params,
    scratch_types=list(_ring_scratch),
)(_ring_pass(_no_scale, use_gidx=False))


def _dense_c_body(p_ref, c_ref, xr_ref, wn_ref, wl_ref, b2_ref, h_ref, y2_ref):
    sums = p_ref[0] + p_ref[1]
    cnt = c_ref[0] + c_ref[1]
    cnt0 = jnp.max(cnt, axis=1, keepdims=True)
    x1 = sums / jnp.maximum(cnt0, 1.0) + xr_ref[...]
    h_ref[...] = jnp.dot(x1, wn_ref[...], preferred_element_type=jnp.float32)
    y2_ref[...] = (jnp.dot(x1, wl_ref[...], preferred_element_type=jnp.float32)
                   + b2_ref[...])


_dense_c = pl.pallas_call(
    _dense_c_body,
    out_shape=[
        jax.ShapeDtypeStruct((_N, _H1), jnp.float32),
        jax.ShapeDtypeStruct((_N, _H2), jnp.float32),
    ],
)


def _dense_e_body(q_ref, y2_ref, o_ref):
    o_ref[...] = q_ref[0] + q_ref[1] + y2_ref[...]


_dense_e = pl.pallas_call(
    _dense_e_body,
    out_shape=jax.ShapeDtypeStruct((_N, _H2), jnp.float32),
)


def kernel(node_features, edge_index, edge_type, edge_norm, basis, att, root,
           bias1, w_nbr, w_lin, bias2):
    src2 = edge_index[0].reshape(_NCHUNK, _CH)
    typ2 = edge_type.reshape(_NCHUNK, _CH)
    dst2 = edge_index[1].reshape(_NCHUNK, _CH)
    norm2 = edge_norm.reshape(_NCHUNK, _CH)
    zsum = jnp.zeros((_ROWS_PER_TILE, _H1), jnp.float32)
    zcnt = jnp.zeros((_ROWS_PER_TILE, _CW), jnp.float32)

    xw, xr = _dense_a(att, basis, node_features, root, bias1.reshape(1, _H1))
    xw_flat = xw.reshape(_R * _N, _H1)
    part1, cnt1 = _sc_pass1(xw_flat, src2, typ2, dst2, norm2, zsum, zcnt)
    h, y2 = _dense_c(part1, cnt1, xr, w_nbr, w_lin, bias2.reshape(1, _H2))
    part2 = _sc_pass2(h, src2, dst2, zsum)
    return _dense_e(part2, y2)
